# baseline (speedup 1.0000x reference)
import functools

import jax
import jax.numpy as jnp
from jax import lax
from jax.experimental import pallas as pl
from jax.experimental.pallas import tpu as pltpu
from jax.experimental.pallas import tpu_sc as plsc

B, F, D = 4096, 26, 16
NW = 32
BPW = B // NW
NCHUNK = 4
CB = BPW // NCHUNK
FD = F * D

_mesh = plsc.VectorSubcoreMesh(core_axis_name="c", subcore_axis_name="s")


@functools.partial(
    pl.kernel,
    mesh=_mesh,
    out_type=jax.ShapeDtypeStruct((B,), jnp.float32),
    compiler_params=pltpu.CompilerParams(needs_layout_passes=False),
    scratch_types=[
        pltpu.VMEM((F, BPW), jnp.int32),
        pltpu.VMEM((F * BPW,), jnp.float32),
        pltpu.VMEM((NCHUNK, CB, FD), jnp.float32),
        pltpu.VMEM((BPW * D,), jnp.float32),
        pltpu.VMEM((BPW,), jnp.float32),
        pltpu.SemaphoreType.DMA,
        pltpu.SemaphoreType.DMA,
        pltpu.SemaphoreType.DMA,
        pltpu.SemaphoreType.DMA,
        pltpu.SemaphoreType.DMA,
    ],
)
def _fm_sc(idx_hbm, emb_hbm, w_hbm, out_hbm, idx_v, rows_v, emb_v, part_v,
           out_v, sem_g, sem_e0, sem_e1, sem_e2, sem_e3):
    wid = lax.axis_index("s") * 2 + lax.axis_index("c")

    pltpu.sync_copy(idx_hbm.at[wid], idx_v)
    sems_e = (sem_e0, sem_e1, sem_e2, sem_e3)
    cps_e = [
        pltpu.async_copy(emb_hbm.at[wid, c], emb_v.at[c], sems_e[c])
        for c in range(NCHUNK)
    ]
    cps_g = [
        pltpu.async_copy(w_hbm.at[idx_v.at[f]], rows_v.at[pl.ds(f * BPW, BPW)],
                         sem_g)
        for f in range(F)
    ]

    for c in range(NCHUNK):
        cps_e[c].wait()

        def body(b, _, c=c):
            v0 = emb_v[c, b, pl.ds(0, D)]
            acc_s = v0
            acc_q = v0 * v0
            for f in range(1, F):
                v = emb_v[c, b, pl.ds(f * D, D)]
                acc_s = acc_s + v
                acc_q = acc_q + v * v
            part_v[pl.ds((c * CB + b) * D, D)] = acc_s * acc_s - acc_q
            return 0

        lax.fori_loop(0, CB, body, 0)

    for cp in cps_g:
        cp.wait()

    iota16 = lax.iota(jnp.int32, 16)
    for g in range(BPW // 16):
        bvec = g * 16 + iota16
        p = bvec * F
        first = plsc.load_gather(rows_v, [p])
        for f in range(1, F):
            p = p + 1
            first = first + plsc.load_gather(rows_v, [p])
        q = bvec * D
        sec = plsc.load_gather(part_v, [q])
        for d in range(1, D):
            q = q + 1
            sec = sec + plsc.load_gather(part_v, [q])
        out_v[pl.ds(g * 16, 16)] = first + 0.5 * sec

    pltpu.sync_copy(out_v, out_hbm.at[pl.ds(wid * BPW, BPW)])


def kernel(sparse_inputs, embed_inputs, w):
    idx = sparse_inputs.astype(jnp.int32).reshape(NW, F, BPW)
    emb = embed_inputs.reshape(NW, NCHUNK, CB, FD)
    wf = w.reshape(-1)
    out = _fm_sc(idx, emb, wf)
    return out.reshape(B, 1)


# speedup vs baseline: 1.7875x; 1.7875x over previous
import functools

import jax
import jax.numpy as jnp
from jax import lax
from jax.experimental import pallas as pl
from jax.experimental.pallas import tpu as pltpu
from jax.experimental.pallas import tpu_sc as plsc

B, F, D = 4096, 26, 16
NW = 32
BPW = B // NW
NL = 16
NGRP = BPW // NL

_mesh = plsc.VectorSubcoreMesh(core_axis_name="c", subcore_axis_name="s")


@functools.partial(
    pl.kernel,
    mesh=_mesh,
    out_type=jax.ShapeDtypeStruct((B,), jnp.float32),
    compiler_params=pltpu.CompilerParams(needs_layout_passes=False),
    scratch_types=[
        pltpu.VMEM((F, BPW), jnp.int32),
        pltpu.VMEM((F * BPW,), jnp.float32),
        pltpu.VMEM((F, D, BPW), jnp.float32),
        pltpu.VMEM((BPW,), jnp.float32),
        pltpu.SemaphoreType.DMA,
        pltpu.SemaphoreType.DMA,
    ],
)
def _fm_sc(idx_hbm, emb_hbm, w_hbm, out_hbm, idx_v, rows_v, emb_v,
           out_v, sem_g, sem_e):
    wid = lax.axis_index("s") * 2 + lax.axis_index("c")
    b0 = wid * BPW

    cp_e = pltpu.async_copy(emb_hbm.at[:, :, pl.ds(b0, BPW)], emb_v, sem_e)
    pltpu.sync_copy(idx_hbm.at[:, pl.ds(b0, BPW)], idx_v)
    cps_g = [
        pltpu.async_copy(w_hbm.at[idx_v.at[f]],
                         rows_v.at[pl.ds(f * BPW, BPW)], sem_g)
        for f in range(F)
    ]
    cp_e.wait()

    for g in range(NGRP):
        gb = g * NL

        def body(d, sec, gb=gb):
            v0 = emb_v[0, d, pl.ds(gb, NL)]
            acc_s = v0
            acc_q = v0 * v0
            for f in range(1, F):
                v = emb_v[f, d, pl.ds(gb, NL)]
                acc_s = acc_s + v
                acc_q = acc_q + v * v
            return sec + (acc_s * acc_s - acc_q)

        sec = lax.fori_loop(0, D, body, jnp.zeros((NL,), jnp.float32))
        out_v[pl.ds(gb, NL)] = 0.5 * sec

    for cp in cps_g:
        cp.wait()

    for g in range(NGRP):
        gb = g * NL
        first = rows_v[pl.ds(gb, NL)]
        for f in range(1, F):
            first = first + rows_v[pl.ds(f * BPW + gb, NL)]
        out_v[pl.ds(gb, NL)] = out_v[pl.ds(gb, NL)] + first

    pltpu.sync_copy(out_v, out_hbm.at[pl.ds(b0, BPW)])


def kernel(sparse_inputs, embed_inputs, w):
    idx = sparse_inputs.astype(jnp.int32).T
    emb = jnp.transpose(embed_inputs, (1, 2, 0))
    out = _fm_sc(idx, emb, w.reshape(-1))
    return out.reshape(B, 1)


# speedup vs baseline: 1.8142x; 1.0149x over previous
import functools

import jax
import jax.numpy as jnp
from jax import lax
from jax.experimental import pallas as pl
from jax.experimental.pallas import tpu as pltpu
from jax.experimental.pallas import tpu_sc as plsc

B, F, D = 4096, 26, 16
NW = 32
BPW = B // NW
NL = 16
NGRP = BPW // NL

_mesh = plsc.VectorSubcoreMesh(core_axis_name="c", subcore_axis_name="s")


@functools.partial(
    pl.kernel,
    mesh=_mesh,
    out_type=jax.ShapeDtypeStruct((B,), jnp.float32),
    compiler_params=pltpu.CompilerParams(needs_layout_passes=False),
    scratch_types=[
        pltpu.VMEM((F, D, BPW), jnp.float32),
        pltpu.VMEM((BPW,), jnp.float32),
        pltpu.SemaphoreType.DMA,
    ],
)
def _fm_dense(emb_hbm, sec_hbm, emb_v, sec_v, sem_e):
    wid = lax.axis_index("s") * 2 + lax.axis_index("c")
    b0 = wid * BPW
    pltpu.async_copy(emb_hbm.at[:, :, pl.ds(b0, BPW)], emb_v, sem_e).wait()

    for g in range(NGRP):
        gb = g * NL

        def body(d, sec, gb=gb):
            v0 = emb_v[0, d, pl.ds(gb, NL)]
            acc_s = v0
            acc_q = v0 * v0
            for f in range(1, F):
                v = emb_v[f, d, pl.ds(gb, NL)]
                acc_s = acc_s + v
                acc_q = acc_q + v * v
            return sec + (acc_s * acc_s - acc_q)

        sec = lax.fori_loop(0, D, body, jnp.zeros((NL,), jnp.float32))
        sec_v[pl.ds(gb, NL)] = 0.5 * sec

    pltpu.sync_copy(sec_v, sec_hbm.at[pl.ds(b0, BPW)])


@functools.partial(
    pl.kernel,
    mesh=_mesh,
    out_type=jax.ShapeDtypeStruct((B,), jnp.float32),
    compiler_params=pltpu.CompilerParams(needs_layout_passes=False),
    scratch_types=[
        pltpu.VMEM((F, BPW), jnp.int32),
        pltpu.VMEM((F * BPW,), jnp.float32),
        pltpu.VMEM((BPW,), jnp.float32),
        pltpu.VMEM((BPW,), jnp.float32),
        pltpu.SemaphoreType.DMA,
        pltpu.SemaphoreType.DMA,
    ],
)
def _fm_gather(idx_hbm, w_hbm, sec_hbm, out_hbm, idx_v, rows_v, sec_v, out_v,
               sem_g, sem_s):
    wid = lax.axis_index("s") * 2 + lax.axis_index("c")
    b0 = wid * BPW

    cp_s = pltpu.async_copy(sec_hbm.at[pl.ds(b0, BPW)], sec_v, sem_s)
    pltpu.sync_copy(idx_hbm.at[:, pl.ds(b0, BPW)], idx_v)
    cps_g = [
        pltpu.async_copy(w_hbm.at[idx_v.at[f]],
                         rows_v.at[pl.ds(f * BPW, BPW)], sem_g)
        for f in range(F)
    ]
    for cp in cps_g:
        cp.wait()
    cp_s.wait()

    for g in range(NGRP):
        gb = g * NL
        first = rows_v[pl.ds(gb, NL)]
        for f in range(1, F):
            first = first + rows_v[pl.ds(f * BPW + gb, NL)]
        out_v[pl.ds(gb, NL)] = sec_v[pl.ds(gb, NL)] + first

    pltpu.sync_copy(out_v, out_hbm.at[pl.ds(b0, BPW)])


def kernel(sparse_inputs, embed_inputs, w):
    idx = sparse_inputs.astype(jnp.int32).T
    emb = jnp.transpose(embed_inputs, (1, 2, 0))
    wf = w.reshape(-1)
    sec = _fm_dense(emb)
    out = _fm_gather(idx, wf, sec)
    return out.reshape(B, 1)
